# Initial kernel scaffold; baseline (speedup 1.0000x reference)
import jax
import jax.numpy as jnp
from jax.experimental import pallas as pl


def kernel(sequence_tensor, span_indices):
    raise NotImplementedError("write your pallas kernel here")


# speedup vs baseline: 1.1322x; 1.1322x over previous
import functools

import jax
import jax.numpy as jnp
from jax import lax
from jax.experimental import pallas as pl
from jax.experimental.pallas import tpu as pltpu
from jax.experimental.pallas import tpu_sc as plsc

B = 4
S = 8192
D = 768
NSPANS = 2048
NROWS = B * NSPANS * 2
ROWS_PER_BATCH = NSPANS * 2

NC = 2
NS = 16
NW = NC * NS
ROWS_PER_W = NROWS // NW
CHUNK = 64
NCHUNK = ROWS_PER_W // CHUNK
LANES = 16

_mesh = plsc.VectorSubcoreMesh(core_axis_name="c", subcore_axis_name="s")


@functools.partial(
    pl.kernel,
    mesh=_mesh,
    out_type=jax.ShapeDtypeStruct((NROWS, D), jnp.float32),
    scratch_types=[
        pltpu.VMEM((ROWS_PER_W,), jnp.int32),
        pltpu.VMEM((CHUNK, D), jnp.float32),
        pltpu.VMEM((CHUNK, D), jnp.float32),
        pltpu.SemaphoreType.DMA,
    ],
)
def _span_gather(table_hbm, idx_hbm, out_hbm, idx_v, buf0, buf1, gsem):
    wid = lax.axis_index("s") * NC + lax.axis_index("c")
    base = wid * ROWS_PER_W

    pltpu.sync_copy(idx_hbm.at[pl.ds(base, ROWS_PER_W)], idx_v)
    offset = (wid // (NW // B)) * S
    for i in range(ROWS_PER_W // LANES):
        sl = pl.ds(i * LANES, LANES)
        idx_v[sl] = idx_v[sl] + offset

    bufs = (buf0, buf1)
    copies = [None, None]
    copies[0] = pltpu.async_copy(
        table_hbm.at[idx_v.at[pl.ds(0, CHUNK)]], bufs[0], gsem)
    for g in range(NCHUNK):
        if g + 1 < NCHUNK:
            copies[(g + 1) % 2] = pltpu.async_copy(
                table_hbm.at[idx_v.at[pl.ds((g + 1) * CHUNK, CHUNK)]],
                bufs[(g + 1) % 2], gsem)
        copies[g % 2].wait()
        pltpu.sync_copy(bufs[g % 2],
                        out_hbm.at[pl.ds(base + g * CHUNK, CHUNK)])


def kernel(sequence_tensor, span_indices):
    table = sequence_tensor.reshape(B * S, D)
    idx = span_indices.reshape(-1)
    out = _span_gather(table, idx)
    return out.reshape(B, NSPANS, 2 * D)


# speedup vs baseline: 1.1419x; 1.0085x over previous
import functools

import jax
import jax.numpy as jnp
from jax import lax
from jax.experimental import pallas as pl
from jax.experimental.pallas import tpu as pltpu
from jax.experimental.pallas import tpu_sc as plsc

B = 4
S = 8192
D = 768
NSPANS = 2048
NROWS = B * NSPANS * 2
ROWS_PER_BATCH = NSPANS * 2

NC = 2
NS = 16
NW = NC * NS
ROWS_PER_W = NROWS // NW
CHUNK = 64
NCHUNK = ROWS_PER_W // CHUNK
LANES = 16

_mesh = plsc.VectorSubcoreMesh(core_axis_name="c", subcore_axis_name="s")


@functools.partial(
    pl.kernel,
    mesh=_mesh,
    out_type=jax.ShapeDtypeStruct((NROWS, D), jnp.float32),
    scratch_types=[
        pltpu.VMEM((ROWS_PER_W,), jnp.int32),
        pltpu.VMEM((CHUNK, D), jnp.float32),
        pltpu.VMEM((CHUNK, D), jnp.float32),
        pltpu.SemaphoreType.DMA,
        pltpu.SemaphoreType.DMA,
    ],
)
def _span_gather(table_hbm, idx_hbm, out_hbm, idx_v, buf0, buf1, gsem, wsem):
    wid = lax.axis_index("s") * NC + lax.axis_index("c")
    base = wid * ROWS_PER_W

    pltpu.sync_copy(idx_hbm.at[pl.ds(base, ROWS_PER_W)], idx_v)
    offset = (wid // (NW // B)) * S
    for i in range(ROWS_PER_W // LANES):
        sl = pl.ds(i * LANES, LANES)
        idx_v[sl] = idx_v[sl] + offset

    bufs = (buf0, buf1)
    copies = [None, None]
    wcopies = [None, None]
    copies[0] = pltpu.async_copy(
        table_hbm.at[idx_v.at[pl.ds(0, CHUNK)]], bufs[0], gsem)
    for g in range(NCHUNK):
        nb = (g + 1) % 2
        if g + 1 < NCHUNK:
            if wcopies[nb] is not None:
                wcopies[nb].wait()
            copies[nb] = pltpu.async_copy(
                table_hbm.at[idx_v.at[pl.ds((g + 1) * CHUNK, CHUNK)]],
                bufs[nb], gsem)
        copies[g % 2].wait()
        wcopies[g % 2] = pltpu.async_copy(
            bufs[g % 2], out_hbm.at[pl.ds(base + g * CHUNK, CHUNK)], wsem)
    wcopies[0].wait()
    wcopies[1].wait()


def kernel(sequence_tensor, span_indices):
    table = sequence_tensor.reshape(B * S, D)
    idx = span_indices.reshape(-1)
    out = _span_gather(table, idx)
    return out.reshape(B, NSPANS, 2 * D)


# speedup vs baseline: 2.2637x; 1.9825x over previous
import functools

import jax
import jax.numpy as jnp
from jax import lax
from jax.experimental import pallas as pl
from jax.experimental.pallas import tpu as pltpu
from jax.experimental.pallas import tpu_sc as plsc

B = 4
S = 8192
D = 768
NSPANS = 2048
NROWS = B * NSPANS * 2

NC = 2
NS = 16
NW = NC * NS
SPANS_PER_W = B * NSPANS // NW
ROWS_PER_W = 2 * SPANS_PER_W
CHUNK = 32
NCHUNK = SPANS_PER_W // CHUNK
LANES = 16
W_PER_BATCH = NW // B

_mesh = plsc.VectorSubcoreMesh(core_axis_name="c", subcore_axis_name="s")


@functools.partial(
    pl.kernel,
    mesh=_mesh,
    out_type=jax.ShapeDtypeStruct((B, NSPANS, 2 * D), jnp.float32),
    scratch_types=[
        pltpu.VMEM((SPANS_PER_W,), jnp.int32),
        pltpu.VMEM((SPANS_PER_W,), jnp.int32),
        pltpu.VMEM((CHUNK, D), jnp.float32),
        pltpu.VMEM((CHUNK, D), jnp.float32),
        pltpu.VMEM((CHUNK, D), jnp.float32),
        pltpu.VMEM((CHUNK, D), jnp.float32),
        pltpu.SemaphoreType.DMA,
        pltpu.SemaphoreType.DMA,
    ],
)
def _span_gather(table_hbm, idxs_hbm, idxe_hbm, out_hbm, idx_s, idx_e,
                 bufs0, bufs1, bufe0, bufe1, gsem, wsem):
    wid = lax.axis_index("s") * NC + lax.axis_index("c")
    b = wid // W_PER_BATCH
    s0 = (wid % W_PER_BATCH) * SPANS_PER_W

    span0 = wid * SPANS_PER_W
    pltpu.sync_copy(idxs_hbm.at[pl.ds(span0, SPANS_PER_W)], idx_s)
    pltpu.sync_copy(idxe_hbm.at[pl.ds(span0, SPANS_PER_W)], idx_e)
    offset = b * S
    for i in range(SPANS_PER_W // LANES):
        sl = pl.ds(i * LANES, LANES)
        idx_s[sl] = idx_s[sl] + offset
        idx_e[sl] = idx_e[sl] + offset

    bufS = (bufs0, bufs1)
    bufE = (bufe0, bufe1)
    gS = [None, None]
    gE = [None, None]
    wS = [None, None]
    wE = [None, None]

    def start_gathers(g, slot):
        gS[slot] = pltpu.async_copy(
            table_hbm.at[idx_s.at[pl.ds(g * CHUNK, CHUNK)]], bufS[slot], gsem)
        gE[slot] = pltpu.async_copy(
            table_hbm.at[idx_e.at[pl.ds(g * CHUNK, CHUNK)]], bufE[slot], gsem)

    start_gathers(0, 0)
    for g in range(NCHUNK):
        slot = g % 2
        nb = (g + 1) % 2
        if g + 1 < NCHUNK:
            if wS[nb] is not None:
                wS[nb].wait()
                wE[nb].wait()
            start_gathers(g + 1, nb)
        gS[slot].wait()
        gE[slot].wait()
        row = pl.ds(s0 + g * CHUNK, CHUNK)
        wS[slot] = pltpu.async_copy(
            bufS[slot], out_hbm.at[b, row, pl.ds(0, D)], wsem)
        wE[slot] = pltpu.async_copy(
            bufE[slot], out_hbm.at[b, row, pl.ds(D, D)], wsem)
    for slot in range(2):
        wS[slot].wait()
        wE[slot].wait()


def kernel(sequence_tensor, span_indices):
    table = sequence_tensor.reshape(B * S, D)
    idx_s = span_indices[..., 0].reshape(-1)
    idx_e = span_indices[..., 1].reshape(-1)
    return _span_gather(table, idx_s, idx_e)
